# Initial kernel scaffold; baseline (speedup 1.0000x reference)
import jax
import jax.numpy as jnp
from jax.experimental import pallas as pl
from jax.experimental.pallas import tpu as pltpu

T = 4096
DIM = 768
EXP = 1536
E = 8

TILE_T = 1024


def _body(x_ref, r_ref, w0_ref, b0_ref, w1_ref, b1_ref, o_ref):
    e = pl.program_id(1)

    @pl.when(e == 0)
    def _():
        o_ref[...] = jnp.zeros_like(o_ref)

    h = jnp.dot(x_ref[...], w0_ref[0], preferred_element_type=jnp.float32)
    h = jax.nn.gelu(h + b0_ref[0], approximate=False)
    y = jnp.dot(h, w1_ref[0], preferred_element_type=jnp.float32)
    y = y + b1_ref[0]
    scale = jax.lax.dynamic_slice_in_dim(r_ref[...], e, 1, axis=1)
    o_ref[...] += y * scale


@jax.jit
def kernel(x, routing_tensor, W0, b0, W1, b1):
    grid = (T // TILE_T, E)
    return pl.pallas_call(
        _body,
        grid=grid,
        in_specs=[
            pl.BlockSpec((TILE_T, DIM), lambda t, e: (t, 0)),
            pl.BlockSpec((TILE_T, E), lambda t, e: (t, 0)),
            pl.BlockSpec((1, DIM, EXP), lambda t, e: (e, 0, 0)),
            pl.BlockSpec((1, 1, EXP), lambda t, e: (e, 0, 0)),
            pl.BlockSpec((1, EXP, DIM), lambda t, e: (e, 0, 0)),
            pl.BlockSpec((1, 1, DIM), lambda t, e: (e, 0, 0)),
        ],
        out_specs=pl.BlockSpec((TILE_T, DIM), lambda t, e: (t, 0)),
        out_shape=jax.ShapeDtypeStruct((T, DIM), jnp.float32),
        compiler_params=pltpu.CompilerParams(
            dimension_semantics=("arbitrary", "arbitrary"),
        ),
    )(x, routing_tensor, W0, b0, W1, b1)


# speedup vs baseline: 3.9682x; 3.9682x over previous
import jax
import jax.numpy as jnp
from jax.experimental import pallas as pl
from jax.experimental.pallas import tpu as pltpu

T = 4096
DIM = 768
EXP = 1536
E = 8

TILE_T = 1024


def _body(x_ref, r_ref, w0_ref, b0_ref, w1_ref, b1_ref, o_ref):
    e = pl.program_id(1)

    @pl.when(e == 0)
    def _():
        o_ref[...] = jnp.zeros_like(o_ref)

    h = jnp.dot(x_ref[...], w0_ref[0], preferred_element_type=jnp.float32)
    h = h + b0_ref[0]
    h = 0.5 * h * (1.0 + jax.lax.erf(h * 0.7071067811865476))
    y = jnp.dot(h, w1_ref[0], preferred_element_type=jnp.float32)
    y = y + b1_ref[0]
    col = jax.lax.broadcasted_iota(jnp.int32, (1, E), 1)
    scale = jnp.sum(jnp.where(col == e, r_ref[...], 0.0), axis=1, keepdims=True)
    o_ref[...] += y * scale


@jax.jit
def kernel(x, routing_tensor, W0, b0, W1, b1):
    grid = (T // TILE_T, E)
    return pl.pallas_call(
        _body,
        grid=grid,
        in_specs=[
            pl.BlockSpec((TILE_T, DIM), lambda t, e: (t, 0)),
            pl.BlockSpec((TILE_T, E), lambda t, e: (t, 0)),
            pl.BlockSpec((1, DIM, EXP), lambda t, e: (e, 0, 0)),
            pl.BlockSpec((1, 1, EXP), lambda t, e: (e, 0, 0)),
            pl.BlockSpec((1, EXP, DIM), lambda t, e: (e, 0, 0)),
            pl.BlockSpec((1, 1, DIM), lambda t, e: (e, 0, 0)),
        ],
        out_specs=pl.BlockSpec((TILE_T, DIM), lambda t, e: (t, 0)),
        out_shape=jax.ShapeDtypeStruct((T, DIM), jnp.float32),
        compiler_params=pltpu.CompilerParams(
            dimension_semantics=("arbitrary", "arbitrary"),
        ),
    )(x, routing_tensor, W0, b0, W1, b1)
